# baseline (speedup 1.0000x reference)
import functools

import jax
import jax.numpy as jnp
from jax import lax
from jax.experimental import pallas as pl
from jax.experimental.pallas import tpu as pltpu
from jax.experimental.pallas import tpu_sc as plsc

DT = 0.02
NC = 2
NS = 16
NW = NC * NS
L = 16
CHUNK = 1600
HALF = CHUNK // 2
UNROLL = 4


def _mesh():
    return plsc.VectorSubcoreMesh(core_axis_name="c", subcore_axis_name="s")


def _sc_params():
    return pltpu.CompilerParams(needs_layout_passes=False)


@functools.lru_cache(maxsize=None)
def _build_fused(n_nodes, n_edges):
    assert n_edges % (NW * CHUNK) == 0
    e_per_w = n_edges // NW
    n_chunks = e_per_w // CHUNK
    assert n_chunks >= 4
    assert CHUNK % (2 * L) == 0 and CHUNK % 8 == 0
    assert n_nodes % L == 0
    half_per_w = e_per_w // 2

    @functools.partial(
        pl.kernel,
        out_type=jax.ShapeDtypeStruct((NW * n_nodes,), jnp.float32),
        mesh=_mesh(),
        scratch_types=[
            pltpu.VMEM((n_nodes,), jnp.float32),
            pltpu.VMEM((CHUNK,), jnp.int32),
            pltpu.VMEM((CHUNK,), jnp.int32),
            pltpu.VMEM((CHUNK,), jnp.float32),
            pltpu.VMEM((CHUNK,), jnp.float32),
            pltpu.VMEM((CHUNK,), jnp.float32),
            pltpu.VMEM((CHUNK,), jnp.float32),
            pltpu.VMEM((CHUNK,), jnp.float32),
            pltpu.VMEM((CHUNK,), jnp.float32),
            pltpu.VMEM((CHUNK,), jnp.int32),
            pltpu.VMEM((CHUNK,), jnp.int32),
            pltpu.VMEM((HALF,), jnp.int32),
            pltpu.VMEM((HALF,), jnp.int32),
            pltpu.HBM((NW * half_per_w,), jnp.int32),
            pltpu.SemaphoreType.DMA,
            pltpu.SemaphoreType.DMA,
            pltpu.SemaphoreType.DMA,
            pltpu.SemaphoreType.DMA,
            pltpu.SemaphoreType.DMA,
            pltpu.SemaphoreType.DMA,
        ],
        compiler_params=_sc_params(),
    )
    def fused(x_hbm, src_hbm, sign_hbm, cnt_hbm, str_hbm, tgt_hbm, part_hbm,
              work_v, src_v0, src_v1, sign_v0, sign_v1, cnt_v0, cnt_v1,
              str_v0, str_v1, tgt_v0, tgt_v1, pk_v0, pk_v1, stash,
              ina_sem0, ina_sem1, outa_sem0, outa_sem1, inb_sem0, inb_sem1):
        src_v = (src_v0, src_v1)
        sign_v = (sign_v0, sign_v1)
        cnt_v = (cnt_v0, cnt_v1)
        str_v = (str_v0, str_v1)
        tgt_v = (tgt_v0, tgt_v1)
        pk_v = (pk_v0, pk_v1)
        ina_sems = (ina_sem0, ina_sem1)
        outa_sems = (outa_sem0, outa_sem1)
        inb_sems = (inb_sem0, inb_sem1)

        cid = lax.axis_index("c")
        sid = lax.axis_index("s")
        wid = sid * NC + cid
        base = wid * e_per_w
        stash_base = wid * half_per_w

        pltpu.sync_copy(x_hbm, work_v)


        def ina_descs(b, c):
            s_all = pl.ds(base + c * CHUNK, CHUNK)
            return (
                pltpu.make_async_copy(src_hbm.at[s_all], src_v[b], ina_sems[b]),
                pltpu.make_async_copy(sign_hbm.at[s_all], sign_v[b], ina_sems[b]),
                pltpu.make_async_copy(cnt_hbm.at[s_all], cnt_v[b], ina_sems[b]),
                pltpu.make_async_copy(str_hbm.at[s_all], str_v[b], ina_sems[b]),
            )

        def outa_desc(b, c):
            s_st = pl.ds(stash_base + c * HALF, HALF)
            return pltpu.make_async_copy(pk_v[b], stash.at[s_st], outa_sems[b])

        def ina_start(b, c):
            for d in ina_descs(b, c):
                d.start()

        def ina_wait(b, c):
            for d in ina_descs(b, c):
                d.wait()

        def compute_a(b):
            sb, gb, cb, tb, pb = (src_v[b], sign_v[b], cnt_v[b], str_v[b],
                                  pk_v[b])

            def vec_body(j, _):
                s0 = pl.ds(2 * j * L, L)
                s1 = pl.ds((2 * j + 1) * L, L)
                v0 = jnp.maximum(plsc.load_gather(work_v, [sb[s0]]), 0.0)
                v1 = jnp.maximum(plsc.load_gather(work_v, [sb[s1]]), 0.0)
                c0 = v0 * (gb[s0] * cb[s0] * jnp.maximum(tb[s0], 0.0))
                c1 = v1 * (gb[s1] * cb[s1] * jnp.maximum(tb[s1], 0.0))
                packed = plsc.pack(c0, c1, format=plsc.PackFormat.INTERLEAVED)
                pb[pl.ds(j * L, L)] = plsc.bitcast(packed, jnp.int32)
                return _

            lax.fori_loop(0, CHUNK // (2 * L), vec_body, None, unroll=UNROLL)

        ina_start(0, 0)
        ina_start(1, 1)
        for b in range(2):
            ina_wait(b, b)
            compute_a(b)
            outa_desc(b, b).start()
            ina_start(b, b + 2)

        def main_a(i2, _):
            for b in range(2):
                c = 2 * i2 + b
                ina_wait(b, c)
                outa_desc(b, c - 2).wait()
                compute_a(b)
                outa_desc(b, c).start()

                @pl.when(c + 2 < n_chunks)
                def _next(b=b, c=c):
                    ina_start(b, c + 2)
            return _

        lax.fori_loop(1, n_chunks // 2, main_a, None)

        if n_chunks % 2 == 1:
            c_tail = n_chunks - 1
            ina_wait(0, c_tail)
            outa_desc(0, c_tail - 2).wait()
            compute_a(0)
            outa_desc(0, c_tail).start()
            outa_desc(1, c_tail - 1).wait()
            outa_desc(0, c_tail).wait()
        else:
            outa_desc(0, n_chunks - 2).wait()
            outa_desc(1, n_chunks - 1).wait()


        zeros = jnp.zeros((L,), jnp.float32)

        def zero_body(i, _):
            work_v[pl.ds(i * L, L)] = zeros
            return _

        lax.fori_loop(0, n_nodes // L, zero_body, None, unroll=8)

        def inb_descs(b, c):
            s_all = pl.ds(base + c * CHUNK, CHUNK)
            s_st = pl.ds(stash_base + c * HALF, HALF)
            return (
                pltpu.make_async_copy(tgt_hbm.at[s_all], tgt_v[b], inb_sems[b]),
                pltpu.make_async_copy(stash.at[s_st], pk_v[b], inb_sems[b]),
            )

        def inb_start(b, c):
            for d in inb_descs(b, c):
                d.start()

        def inb_wait(b, c):
            for d in inb_descs(b, c):
                d.wait()

        def compute_b(b):
            tb, pb = tgt_v[b], pk_v[b]

            def vec_body(j, _):
                s0 = pl.ds(2 * j * L, L)
                s1 = pl.ds((2 * j + 1) * L, L)
                packed = plsc.bitcast(pb[pl.ds(j * L, L)], jnp.bfloat16)
                c0, c1 = plsc.unpack(packed, format=plsc.PackFormat.INTERLEAVED)
                plsc.addupdate_scatter(work_v, [tb[s0]], c0)
                plsc.addupdate_scatter(work_v, [tb[s1]], c1)
                return _

            lax.fori_loop(0, CHUNK // (2 * L), vec_body, None, unroll=UNROLL)

        inb_start(0, 0)
        inb_start(1, 1)

        def main_b(i2, _):
            for b in range(2):
                c = 2 * i2 + b
                inb_wait(b, c)
                compute_b(b)

                @pl.when(c + 2 < n_chunks)
                def _next(b=b, c=c):
                    inb_start(b, c + 2)
            return _

        lax.fori_loop(0, n_chunks // 2, main_b, None)

        if n_chunks % 2 == 1:
            c_tail = n_chunks - 1
            inb_wait(0, c_tail)
            compute_b(0)

        pltpu.sync_copy(work_v, part_hbm.at[pl.ds(wid * n_nodes, n_nodes)])

    return fused


def _epilogue_body(x_ref, bias_ref, tau_ref, part_ref, o_ref):
    summed = jnp.sum(part_ref[...], axis=0)
    x = x_ref[...]
    o_ref[...] = x + DT * ((-x + bias_ref[...] + summed) / tau_ref[...])


def kernel(x, source_indices, target_indices, sign, syn_count, syn_strength,
           bias, time_const):
    n_nodes = x.shape[0]
    n_edges = source_indices.shape[0]

    fused = _build_fused(n_nodes, n_edges)
    partials = fused(x, source_indices.astype(jnp.int32), sign, syn_count,
                     syn_strength, target_indices.astype(jnp.int32))
    partials = partials.reshape(NW, n_nodes)

    x_new = pl.pallas_call(
        _epilogue_body,
        out_shape=jax.ShapeDtypeStruct((n_nodes,), jnp.float32),
    )(x, bias, time_const, partials)
    return x_new


# speedup vs baseline: 1.1132x; 1.1132x over previous
import functools

import jax
import jax.numpy as jnp
from jax import lax
from jax.experimental import pallas as pl
from jax.experimental.pallas import tpu as pltpu
from jax.experimental.pallas import tpu_sc as plsc

DT = 0.02
NC = 2
NS = 16
NW = NC * NS
L = 16
CHUNK = 1600
HALF = CHUNK // 2
NBUF = 3
UNROLL = 4


def _mesh():
    return plsc.VectorSubcoreMesh(core_axis_name="c", subcore_axis_name="s")


def _sc_params():
    return pltpu.CompilerParams(needs_layout_passes=False)


@functools.lru_cache(maxsize=None)
def _build_fused(n_nodes, n_edges):
    assert n_edges % (NW * CHUNK) == 0
    e_per_w = n_edges // NW
    n_chunks = e_per_w // CHUNK
    n_groups = n_chunks // NBUF
    n_rem = n_chunks % NBUF
    assert n_chunks >= 2 * NBUF
    assert CHUNK % (2 * L) == 0 and CHUNK % 8 == 0
    assert n_nodes % L == 0
    half_per_w = e_per_w // 2

    vmem = [
        pltpu.VMEM((n_nodes,), jnp.float32),
    ]
    vmem += [pltpu.VMEM((CHUNK,), jnp.int32) for _ in range(NBUF)]
    vmem += [pltpu.VMEM((CHUNK,), jnp.float32) for _ in range(NBUF)]
    vmem += [pltpu.VMEM((CHUNK,), jnp.float32) for _ in range(NBUF)]
    vmem += [pltpu.VMEM((CHUNK,), jnp.float32) for _ in range(NBUF)]
    vmem += [pltpu.VMEM((CHUNK,), jnp.int32) for _ in range(NBUF)]
    vmem += [pltpu.VMEM((HALF,), jnp.int32) for _ in range(NBUF)]
    scratch = vmem + [
        pltpu.HBM((NW * half_per_w,), jnp.int32),
    ]
    scratch += [pltpu.SemaphoreType.DMA for _ in range(3 * NBUF)]

    @functools.partial(
        pl.kernel,
        out_type=jax.ShapeDtypeStruct((NW * n_nodes,), jnp.float32),
        mesh=_mesh(),
        scratch_types=scratch,
        compiler_params=_sc_params(),
    )
    def fused(x_hbm, src_hbm, sign_hbm, cnt_hbm, str_hbm, tgt_hbm, part_hbm,
              work_v, *rest):
        src_v = rest[0:NBUF]
        sign_v = rest[NBUF:2 * NBUF]
        cnt_v = rest[2 * NBUF:3 * NBUF]
        str_v = rest[3 * NBUF:4 * NBUF]
        tgt_v = rest[4 * NBUF:5 * NBUF]
        pk_v = rest[5 * NBUF:6 * NBUF]
        stash = rest[6 * NBUF]
        ina_sems = rest[6 * NBUF + 1:6 * NBUF + 1 + NBUF]
        outa_sems = rest[6 * NBUF + 1 + NBUF:6 * NBUF + 1 + 2 * NBUF]
        inb_sems = rest[6 * NBUF + 1 + 2 * NBUF:6 * NBUF + 1 + 3 * NBUF]

        cid = lax.axis_index("c")
        sid = lax.axis_index("s")
        wid = sid * NC + cid
        base = wid * e_per_w
        stash_base = wid * half_per_w


        def ina_descs(b, c):
            s_all = pl.ds(base + c * CHUNK, CHUNK)
            return (
                pltpu.make_async_copy(src_hbm.at[s_all], src_v[b], ina_sems[b]),
                pltpu.make_async_copy(sign_hbm.at[s_all], sign_v[b], ina_sems[b]),
                pltpu.make_async_copy(cnt_hbm.at[s_all], cnt_v[b], ina_sems[b]),
                pltpu.make_async_copy(str_hbm.at[s_all], str_v[b], ina_sems[b]),
            )

        def outa_desc(b, c):
            s_st = pl.ds(stash_base + c * HALF, HALF)
            return pltpu.make_async_copy(pk_v[b], stash.at[s_st], outa_sems[b])

        def ina_start(b, c):
            for d in ina_descs(b, c):
                d.start()

        def ina_wait(b, c):
            for d in ina_descs(b, c):
                d.wait()

        def compute_a(b):
            sb, gb, cb, tb, pb = (src_v[b], sign_v[b], cnt_v[b], str_v[b],
                                  pk_v[b])

            def vec_body(j, _):
                s0 = pl.ds(2 * j * L, L)
                s1 = pl.ds((2 * j + 1) * L, L)
                v0 = jnp.maximum(plsc.load_gather(work_v, [sb[s0]]), 0.0)
                v1 = jnp.maximum(plsc.load_gather(work_v, [sb[s1]]), 0.0)
                c0 = v0 * (gb[s0] * cb[s0] * jnp.maximum(tb[s0], 0.0))
                c1 = v1 * (gb[s1] * cb[s1] * jnp.maximum(tb[s1], 0.0))
                packed = plsc.pack(c0, c1, format=plsc.PackFormat.INTERLEAVED)
                pb[pl.ds(j * L, L)] = plsc.bitcast(packed, jnp.int32)
                return _

            lax.fori_loop(0, CHUNK // (2 * L), vec_body, None, unroll=UNROLL)

        for b in range(NBUF):
            ina_start(b, b)
        pltpu.sync_copy(x_hbm, work_v)

        for b in range(NBUF):
            ina_wait(b, b)
            compute_a(b)
            outa_desc(b, b).start()
            ina_start(b, b + NBUF)

        def main_a(g, _):
            for b in range(NBUF):
                c = g * NBUF + b
                ina_wait(b, c)
                outa_desc(b, c - NBUF).wait()
                compute_a(b)
                outa_desc(b, c).start()

                @pl.when(c + NBUF < n_chunks)
                def _next(b=b, c=c):
                    ina_start(b, c + NBUF)
            return _

        lax.fori_loop(1, n_groups, main_a, None)

        for b in range(n_rem):
            c = n_groups * NBUF + b
            ina_wait(b, c)
            outa_desc(b, c - NBUF).wait()
            compute_a(b)
            outa_desc(b, c).start()

        for k in range(NBUF):
            c = n_chunks - NBUF + k
            outa_desc(c % NBUF, c).wait()


        def inb_descs(b, c):
            s_all = pl.ds(base + c * CHUNK, CHUNK)
            s_st = pl.ds(stash_base + c * HALF, HALF)
            return (
                pltpu.make_async_copy(tgt_hbm.at[s_all], tgt_v[b], inb_sems[b]),
                pltpu.make_async_copy(stash.at[s_st], pk_v[b], inb_sems[b]),
            )

        def inb_start(b, c):
            for d in inb_descs(b, c):
                d.start()

        def inb_wait(b, c):
            for d in inb_descs(b, c):
                d.wait()

        def compute_b(b):
            tb, pb = tgt_v[b], pk_v[b]

            def vec_body(j, _):
                s0 = pl.ds(2 * j * L, L)
                s1 = pl.ds((2 * j + 1) * L, L)
                packed = plsc.bitcast(pb[pl.ds(j * L, L)], jnp.bfloat16)
                c0, c1 = plsc.unpack(packed, format=plsc.PackFormat.INTERLEAVED)
                plsc.addupdate_scatter(work_v, [tb[s0]], c0)
                plsc.addupdate_scatter(work_v, [tb[s1]], c1)
                return _

            lax.fori_loop(0, CHUNK // (2 * L), vec_body, None, unroll=UNROLL)

        for b in range(NBUF):
            inb_start(b, b)

        zeros = jnp.zeros((L,), jnp.float32)

        def zero_body(i, _):
            work_v[pl.ds(i * L, L)] = zeros
            return _

        lax.fori_loop(0, n_nodes // L, zero_body, None, unroll=8)

        def main_b(g, _):
            for b in range(NBUF):
                c = g * NBUF + b
                inb_wait(b, c)
                compute_b(b)

                @pl.when(c + NBUF < n_chunks)
                def _next(b=b, c=c):
                    inb_start(b, c + NBUF)
            return _

        lax.fori_loop(0, n_groups, main_b, None)

        for b in range(n_rem):
            c = n_groups * NBUF + b
            inb_wait(b, c)
            compute_b(b)

        pltpu.sync_copy(work_v, part_hbm.at[pl.ds(wid * n_nodes, n_nodes)])

    return fused


def _epilogue_body(x_ref, bias_ref, tau_ref, part_ref, o_ref):
    summed = jnp.sum(part_ref[...], axis=0)
    x = x_ref[...]
    o_ref[...] = x + DT * ((-x + bias_ref[...] + summed) / tau_ref[...])


def kernel(x, source_indices, target_indices, sign, syn_count, syn_strength,
           bias, time_const):
    n_nodes = x.shape[0]
    n_edges = source_indices.shape[0]

    fused = _build_fused(n_nodes, n_edges)
    partials = fused(x, source_indices.astype(jnp.int32), sign, syn_count,
                     syn_strength, target_indices.astype(jnp.int32))
    partials = partials.reshape(NW, n_nodes)

    x_new = pl.pallas_call(
        _epilogue_body,
        out_shape=jax.ShapeDtypeStruct((n_nodes,), jnp.float32),
    )(x, bias, time_const, partials)
    return x_new
